# baseline (speedup 1.0000x reference)
import functools

import jax
import jax.numpy as jnp
from jax.experimental import pallas as pl
from jax.experimental.pallas import tpu as pltpu

D_MODEL = 768
NUM_EXPERTS = 8
TOP_K = 2
BLOCK = 8192
NSPLIT = 1
SUB = BLOCK // NSPLIT


def _router_block(*refs):
    x_refs = refs[:NSPLIT]
    w_ref, b_ref, mix_ref, probs_ref, idx_ref, tw_ref = refs[NSPLIT:]
    w = w_ref[...]
    parts = [
        jax.lax.dot_general(
            w, xr[...], (((1,), (1,)), ((), ())),
            preferred_element_type=jnp.float32,
        )
        for xr in x_refs
    ]
    logits = jnp.concatenate(parts, axis=1) + b_ref[...].T

    m = jnp.max(logits, axis=0, keepdims=True)
    e = jnp.exp(logits - m)
    s = jnp.sum(e, axis=0, keepdims=True)
    probs = e * (1.0 / s)

    erows = jax.lax.broadcasted_iota(jnp.int32, e.shape, 0)
    v1 = jnp.max(e, axis=0, keepdims=True)
    i1 = jnp.min(jnp.where(e == v1, erows, NUM_EXPERTS), axis=0, keepdims=True)
    masked = jnp.where(erows == i1, -1.0, e)
    v2 = jnp.max(masked, axis=0, keepdims=True)
    i2 = jnp.min(jnp.where(masked == v2, erows, NUM_EXPERTS), axis=0, keepdims=True)

    inv = 1.0 / (v1 + v2)
    w1 = v1 * inv
    w2 = v2 * inv

    zero = jnp.zeros_like(e)
    mixing = jnp.where(erows == i1, w1, zero) + jnp.where(erows == i2, w2, zero)

    mix_ref[...] = mixing
    probs_ref[...] = probs
    idx_ref[...] = jnp.concatenate([i1, i2], axis=0)
    tw_ref[...] = jnp.concatenate([w1, w2], axis=0)


@functools.partial(jax.jit, static_argnames=())
def kernel(x, W, b):
    n, d = x.shape
    e = W.shape[0]
    b2 = b.reshape(1, e)
    grid = (n // BLOCK,)
    out = pl.pallas_call(
        _router_block,
        grid=grid,
        in_specs=[
            pl.BlockSpec((SUB, d), functools.partial(
                lambda j, i: (NSPLIT * i + j, 0), j))
            for j in range(NSPLIT)
        ] + [
            pl.BlockSpec((e, d), lambda i: (0, 0)),
            pl.BlockSpec((1, e), lambda i: (0, 0)),
        ],
        out_specs=[
            pl.BlockSpec((e, BLOCK), lambda i: (0, i)),
            pl.BlockSpec((e, BLOCK), lambda i: (0, i)),
            pl.BlockSpec((TOP_K, BLOCK), lambda i: (0, i)),
            pl.BlockSpec((TOP_K, BLOCK), lambda i: (0, i)),
        ],
        out_shape=[
            jax.ShapeDtypeStruct((e, n), jnp.float32),
            jax.ShapeDtypeStruct((e, n), jnp.float32),
            jax.ShapeDtypeStruct((TOP_K, n), jnp.int32),
            jax.ShapeDtypeStruct((TOP_K, n), jnp.float32),
        ],
        compiler_params=pltpu.CompilerParams(
            dimension_semantics=("parallel",),
        ),
    )(*([x] * NSPLIT), W, b2)
    mix_t, probs_t, idx_t, tw_t = out
    return (mix_t.T, probs_t.T, idx_t.T, tw_t.T)


# speedup vs baseline: 1.0685x; 1.0685x over previous
import functools

import jax
import jax.numpy as jnp
from jax.experimental import pallas as pl
from jax.experimental.pallas import tpu as pltpu

D_MODEL = 768
NUM_EXPERTS = 8
TOP_K = 2
BLOCK = 4096


def _router_block(x_ref, w_ref, b_ref, mix_ref, probs_ref, idx_ref, tw_ref):
    w = w_ref[...]
    logits = jax.lax.dot_general(
        w, x_ref[...], (((1,), (1,)), ((), ())),
        preferred_element_type=jnp.float32,
    ) + b_ref[...].T

    m = jnp.max(logits, axis=0, keepdims=True)
    e = jnp.exp(logits - m)
    s = jnp.sum(e, axis=0, keepdims=True)
    probs = e * (1.0 / s)

    erows = jax.lax.broadcasted_iota(jnp.int32, e.shape, 0)
    v1 = jnp.max(e, axis=0, keepdims=True)
    i1 = jnp.min(jnp.where(e == v1, erows, NUM_EXPERTS), axis=0, keepdims=True)
    masked = jnp.where(erows == i1, -1.0, e)
    v2 = jnp.max(masked, axis=0, keepdims=True)
    i2 = jnp.min(jnp.where(masked == v2, erows, NUM_EXPERTS), axis=0, keepdims=True)

    inv = 1.0 / (v1 + v2)
    w1 = v1 * inv
    w2 = v2 * inv

    zero = jnp.zeros_like(e)
    mixing = jnp.where(erows == i1, w1, zero) + jnp.where(erows == i2, w2, zero)

    mix_ref[...] = mixing
    probs_ref[...] = probs
    idx_ref[...] = jnp.concatenate([i1, i2], axis=0)
    tw_ref[...] = jnp.concatenate([w1, w2], axis=0)


@functools.partial(jax.jit, static_argnames=())
def kernel(x, W, b):
    n, d = x.shape
    e = W.shape[0]
    b2 = b.reshape(1, e)
    out = pl.pallas_call(
        _router_block,
        grid=(n // BLOCK,),
        in_specs=[
            pl.BlockSpec((BLOCK, d), lambda i: (i, 0)),
            pl.BlockSpec((e, d), lambda i: (0, 0)),
            pl.BlockSpec((1, e), lambda i: (0, 0)),
        ],
        out_specs=[
            pl.BlockSpec((e, BLOCK), lambda i: (0, i)),
            pl.BlockSpec((e, BLOCK), lambda i: (0, i)),
            pl.BlockSpec((TOP_K, BLOCK), lambda i: (0, i)),
            pl.BlockSpec((TOP_K, BLOCK), lambda i: (0, i)),
        ],
        out_shape=[
            jax.ShapeDtypeStruct((e, n), jnp.float32),
            jax.ShapeDtypeStruct((e, n), jnp.float32),
            jax.ShapeDtypeStruct((TOP_K, n), jnp.int32),
            jax.ShapeDtypeStruct((TOP_K, n), jnp.float32),
        ],
        compiler_params=pltpu.CompilerParams(
            dimension_semantics=("parallel",),
        ),
    )(x, W, b2)
    mix_t, probs_t, idx_t, tw_t = out
    return (mix_t.T, probs_t.T, idx_t.T, tw_t.T)
